# baseline (speedup 1.0000x reference)
import jax
import jax.numpy as jnp
from jax.experimental import pallas as pl
from jax.experimental.pallas import tpu as pltpu

_MARGIN = 1.0
_EPS = 1e-6
_BLK = 1024
_GRID = 2


def _triplet_loss_kernel(a_ref, p_ref, n_ref, out_ref):
    A = a_ref[:, :]
    R, D = A.shape
    N = n_ref[:, :]
    C = N.shape[0]

    row_term = jnp.sum(A * A + (2.0 * _EPS) * A, axis=1, keepdims=True)
    col_term = jnp.sum(N * N - (2.0 * _EPS) * N, axis=1, keepdims=True)
    ones_r = jnp.ones((R, 1), dtype=jnp.float32)
    ones_c = jnp.ones((C, 1), dtype=jnp.float32)
    a_aug = jnp.concatenate([A, ones_r, row_term], axis=1).astype(jnp.bfloat16)
    n_aug = jnp.concatenate([N * -2.0, col_term, ones_c],
                            axis=1).astype(jnp.bfloat16)

    best = jnp.full((R, 1), jnp.inf, dtype=jnp.float32)
    for b in range(C // _BLK):
        nb = jax.lax.slice(n_aug, (b * _BLK, 0), ((b + 1) * _BLK, D + 2))
        z = jax.lax.dot_general(a_aug, nb, (((1,), (1,)), ((), ())),
                                preferred_element_type=jnp.float32)
        best = jnp.minimum(best, jnp.min(z, axis=1, keepdims=True))

    dn = jnp.sqrt(jnp.maximum(best + D * _EPS * _EPS, 0.0))
    diff = A - p_ref[:, :] + _EPS
    dp = jnp.sqrt(jnp.sum(diff * diff, axis=1, keepdims=True))
    losses = jnp.maximum(dp - dn + _MARGIN, 0.0)
    out_ref[0, :, :] = jnp.sum(losses, axis=0, keepdims=True)


def kernel(anchor, positive, negative):
    n_rows = anchor.shape[0]
    blk_rows = n_rows // _GRID
    partial = pl.pallas_call(
        _triplet_loss_kernel,
        grid=(_GRID,),
        in_specs=[
            pl.BlockSpec((blk_rows, anchor.shape[1]), lambda i: (i, 0)),
            pl.BlockSpec((blk_rows, anchor.shape[1]), lambda i: (i, 0)),
            pl.BlockSpec(negative.shape, lambda i: (0, 0)),
        ],
        out_specs=pl.BlockSpec((1, 1, 1), lambda i: (i, 0, 0)),
        out_shape=jax.ShapeDtypeStruct((_GRID, 1, 1), jnp.float32),
        compiler_params=pltpu.CompilerParams(
            dimension_semantics=("parallel",),
        ),
    )(anchor, positive, negative)
    return jnp.sum(partial) / n_rows


# speedup vs baseline: 1.1355x; 1.1355x over previous
import jax
import jax.numpy as jnp
from jax.experimental import pallas as pl
from jax.experimental.pallas import tpu as pltpu

_MARGIN = 1.0
_EPS = 1e-6
_BLK = 2048


def _triplet_loss_kernel(a_ref, p_ref, n_ref, out_ref):
    A = a_ref[:, :]
    R, D = A.shape
    N = n_ref[:, :]
    C = N.shape[0]

    a_term = jnp.sum(A * A + (2.0 * _EPS) * A, axis=1, keepdims=True)
    n_term = jnp.sum(N * N - (2.0 * _EPS) * N, axis=1, keepdims=True)
    ones_r = jnp.ones((R, 1), dtype=jnp.float32)
    ones_c = jnp.ones((C, 1), dtype=jnp.float32)
    a_aug = jnp.concatenate([A, ones_r, a_term], axis=1).astype(jnp.bfloat16)
    n_aug = jnp.concatenate([N * -2.0, n_term, ones_c],
                            axis=1).astype(jnp.bfloat16)

    best8 = jnp.full((8, R), jnp.inf, dtype=jnp.float32)
    for b in range(C // _BLK):
        nb = jax.lax.slice(n_aug, (b * _BLK, 0), ((b + 1) * _BLK, D + 2))
        z = jax.lax.dot_general(nb, a_aug, (((1,), (1,)), ((), ())),
                                preferred_element_type=jnp.float32)
        best8 = jnp.minimum(best8, jnp.min(z.reshape(_BLK // 8, 8, R), axis=0))
    best = jnp.min(best8, axis=0, keepdims=True)

    dn = jnp.sqrt(jnp.maximum(best + D * _EPS * _EPS, 0.0))
    diff = A - p_ref[:, :] + _EPS
    ones_row = jnp.ones((1, D), dtype=jnp.float32)
    dp2 = jax.lax.dot_general(ones_row, diff * diff, (((1,), (1,)), ((), ())),
                              preferred_element_type=jnp.float32)
    losses = jnp.maximum(jnp.sqrt(dp2) - dn + _MARGIN, 0.0)
    out_ref[:, :] = jnp.sum(losses, axis=1, keepdims=True) / R


def kernel(anchor, positive, negative):
    out = pl.pallas_call(
        _triplet_loss_kernel,
        out_shape=jax.ShapeDtypeStruct((1, 1), jnp.float32),
    )(anchor, positive, negative)
    return out[0, 0]


# speedup vs baseline: 1.1543x; 1.0165x over previous
import jax
import jax.numpy as jnp
from jax.experimental import pallas as pl
from jax.experimental.pallas import tpu as pltpu

_MARGIN = 1.0
_EPS = 1e-6
_BLK = 2048


def _triplet_loss_kernel(a_hbm, p_hbm, n_hbm, out_ref,
                         a_v, p_v, n_v, sem_a, sem_p, sem_n):
    cp_a = pltpu.make_async_copy(a_hbm, a_v, sem_a)
    cp_p = pltpu.make_async_copy(p_hbm, p_v, sem_p)
    cp_n = pltpu.make_async_copy(n_hbm, n_v, sem_n)
    cp_a.start()
    cp_n.start()
    cp_p.start()
    cp_a.wait()
    cp_n.wait()

    A = a_v[:, :]
    R, D = A.shape
    N = n_v[:, :]
    C = N.shape[0]

    a_term = jnp.sum(A * A + (2.0 * _EPS) * A, axis=1, keepdims=True)
    n_term = jnp.sum(N * N - (2.0 * _EPS) * N, axis=1, keepdims=True)
    ones_r = jnp.ones((R, 1), dtype=jnp.float32)
    ones_c = jnp.ones((C, 1), dtype=jnp.float32)
    a_aug = jnp.concatenate([A, ones_r, a_term], axis=1).astype(jnp.bfloat16)
    n_aug = jnp.concatenate([N * -2.0, n_term, ones_c],
                            axis=1).astype(jnp.bfloat16)

    best8 = jnp.full((8, R), jnp.inf, dtype=jnp.float32)
    for b in range(C // _BLK):
        nb = jax.lax.slice(n_aug, (b * _BLK, 0), ((b + 1) * _BLK, D + 2))
        z = jax.lax.dot_general(nb, a_aug, (((1,), (1,)), ((), ())),
                                preferred_element_type=jnp.float32)
        best8 = jnp.minimum(best8, jnp.min(z.reshape(_BLK // 8, 8, R), axis=0))
    best = jnp.min(best8, axis=0, keepdims=True)

    dn = jnp.sqrt(jnp.maximum(best + D * _EPS * _EPS, 0.0))
    cp_p.wait()
    diff = A - p_v[:, :] + _EPS
    ones_row = jnp.ones((1, D), dtype=jnp.float32)
    dp2 = jax.lax.dot_general(ones_row, diff * diff, (((1,), (1,)), ((), ())),
                              preferred_element_type=jnp.float32)
    losses = jnp.maximum(jnp.sqrt(dp2) - dn + _MARGIN, 0.0)
    out_ref[:, :] = jnp.sum(losses, axis=1, keepdims=True) / R


def kernel(anchor, positive, negative):
    out = pl.pallas_call(
        _triplet_loss_kernel,
        in_specs=[pl.BlockSpec(memory_space=pltpu.MemorySpace.HBM)] * 3,
        out_shape=jax.ShapeDtypeStruct((1, 1), jnp.float32),
        scratch_shapes=[
            pltpu.VMEM(anchor.shape, jnp.float32),
            pltpu.VMEM(positive.shape, jnp.float32),
            pltpu.VMEM(negative.shape, jnp.float32),
            pltpu.SemaphoreType.DMA,
            pltpu.SemaphoreType.DMA,
            pltpu.SemaphoreType.DMA,
        ],
    )(anchor, positive, negative)
    return out[0, 0]
